# baseline (speedup 1.0000x reference)
import functools

import jax
import jax.numpy as jnp
from jax import lax
from jax.experimental import pallas as pl
from jax.experimental.pallas import tpu as pltpu
from jax.experimental.pallas import tpu_sc as plsc

BATCH = 16384
DIM = 64
NUM_CORES = 2
NUM_SUBCORES = 16
NUM_WORKERS = NUM_CORES * NUM_SUBCORES
B_PER_W = BATCH // NUM_WORKERS
L = 16

_mesh = plsc.VectorSubcoreMesh(core_axis_name="c", subcore_axis_name="s")


@functools.partial(
    pl.kernel,
    mesh=_mesh,
    compiler_params=pltpu.CompilerParams(needs_layout_passes=False),
    out_type=(
        jax.ShapeDtypeStruct((BATCH * DIM,), jnp.float32),
        jax.ShapeDtypeStruct((BATCH,), jnp.int32),
    ),
    scratch_types=[
        pltpu.VMEM((B_PER_W,), jnp.int32),
        pltpu.VMEM((B_PER_W,), jnp.int32),
        pltpu.VMEM((B_PER_W, 2 * DIM), jnp.float32),
        pltpu.VMEM((B_PER_W * DIM,), jnp.float32),
        pltpu.VMEM((B_PER_W,), jnp.int32),
        pltpu.SemaphoreType.DMA,
        pltpu.SemaphoreType.DMA,
    ],
)
def _gather_kernel(nid_hbm, mem2_hbm, last_hbm, mem_out_hbm, last_out_hbm,
                   idx_v, pair_v, buf_v, out_v, last_v, sem_rows, sem_last):
    wid = lax.axis_index("s") * NUM_CORES + lax.axis_index("c")
    base = wid * B_PER_W
    pltpu.sync_copy(nid_hbm.at[pl.ds(base, B_PER_W)], idx_v)

    def _pair(b):
        v = idx_v[pl.ds(b * L, L)]
        pair_v[pl.ds(b * L, L)] = lax.shift_right_logical(v, 1)
    pl.loop(0, B_PER_W // L)(_pair)

    cp_rows = pltpu.async_copy(mem2_hbm.at[pair_v], buf_v, sem_rows)
    cp_last = pltpu.async_copy(last_hbm.at[idx_v], last_v, sem_last)
    cp_rows.wait()

    iota = lax.iota(jnp.int32, L)

    def _extract(r):
        r_vec = jnp.full((L,), r, jnp.int32)
        idxs = plsc.load_gather(idx_v, [r_vec])
        col0 = (idxs & 1) * DIM + iota
        for c in range(DIM // L):
            v = plsc.load_gather(buf_v, [r_vec, col0 + c * L])
            out_v[pl.ds(r * DIM + c * L, L)] = v
    pl.loop(0, B_PER_W)(_extract)

    pltpu.sync_copy(out_v, mem_out_hbm.at[pl.ds(base * DIM, B_PER_W * DIM)])
    cp_last.wait()
    pltpu.sync_copy(last_v, last_out_hbm.at[pl.ds(base, B_PER_W)])


def kernel(n_id, memory, last_update):
    mem2 = memory.reshape(500_000, 2 * DIM)
    mem_flat, last_out = _gather_kernel(
        n_id.astype(jnp.int32), mem2, last_update)
    return (mem_flat.reshape(BATCH, DIM), last_out,
            jnp.array(0, dtype=jnp.int32))


# speedup vs baseline: 1.0011x; 1.0011x over previous
import functools

import jax
import jax.numpy as jnp
from jax import lax
from jax.experimental import pallas as pl
from jax.experimental.pallas import tpu as pltpu
from jax.experimental.pallas import tpu_sc as plsc

BATCH = 16384
DIM = 64
NUM_CORES = 2
NUM_SUBCORES = 16
NUM_WORKERS = NUM_CORES * NUM_SUBCORES
B_PER_W = BATCH // NUM_WORKERS
L = 16

_mesh = plsc.VectorSubcoreMesh(core_axis_name="c", subcore_axis_name="s")


@functools.partial(
    pl.kernel,
    mesh=_mesh,
    compiler_params=pltpu.CompilerParams(
        needs_layout_passes=False, use_tc_tiling_on_sc=True),
    out_type=(
        jax.ShapeDtypeStruct((BATCH * DIM,), jnp.float32),
        jax.ShapeDtypeStruct((BATCH,), jnp.int32),
    ),
    scratch_types=[
        pltpu.VMEM((B_PER_W,), jnp.int32),
        pltpu.VMEM((B_PER_W,), jnp.int32),
        pltpu.VMEM((B_PER_W, 2 * DIM), jnp.float32),
        pltpu.VMEM((B_PER_W * DIM,), jnp.float32),
        pltpu.VMEM((B_PER_W,), jnp.int32),
        pltpu.SemaphoreType.DMA,
        pltpu.SemaphoreType.DMA,
    ],
)
def _gather_kernel(nid_hbm, mem2_hbm, last_hbm, mem_out_hbm, last_out_hbm,
                   idx_v, pair_v, buf_v, out_v, last_v, sem_rows, sem_last):
    wid = lax.axis_index("s") * NUM_CORES + lax.axis_index("c")
    base = wid * B_PER_W
    pltpu.sync_copy(nid_hbm.at[pl.ds(base, B_PER_W)], idx_v)

    def _pair(b):
        v = idx_v[pl.ds(b * L, L)]
        pair_v[pl.ds(b * L, L)] = lax.shift_right_logical(v, 1)
    pl.loop(0, B_PER_W // L)(_pair)

    cp_rows = pltpu.async_copy(mem2_hbm.at[pair_v], buf_v, sem_rows)
    cp_last = pltpu.async_copy(last_hbm.at[idx_v], last_v, sem_last)
    cp_rows.wait()

    iota = lax.iota(jnp.int32, L)

    def _extract(r):
        r_vec = jnp.full((L,), r, jnp.int32)
        idxs = plsc.load_gather(idx_v, [r_vec])
        col0 = (idxs & 1) * DIM + iota
        for c in range(DIM // L):
            v = plsc.load_gather(buf_v, [r_vec, col0 + c * L])
            out_v[pl.ds(r * DIM + c * L, L)] = v
    pl.loop(0, B_PER_W)(_extract)

    pltpu.sync_copy(out_v, mem_out_hbm.at[pl.ds(base * DIM, B_PER_W * DIM)])
    cp_last.wait()
    pltpu.sync_copy(last_v, last_out_hbm.at[pl.ds(base, B_PER_W)])


def kernel(n_id, memory, last_update):
    mem2 = memory.reshape(500_000, 2 * DIM)
    mem_flat, last_out = _gather_kernel(
        n_id.astype(jnp.int32), mem2, last_update)
    return (mem_flat.reshape(BATCH, DIM), last_out,
            jnp.array(0, dtype=jnp.int32))


# speedup vs baseline: 1.1212x; 1.1200x over previous
import functools

import jax
import jax.numpy as jnp
from jax import lax
from jax.experimental import pallas as pl
from jax.experimental.pallas import tpu as pltpu
from jax.experimental.pallas import tpu_sc as plsc

BATCH = 16384
DIM = 64
NUM_CORES = 2
NUM_SUBCORES = 16
NUM_WORKERS = NUM_CORES * NUM_SUBCORES
B_PER_W = BATCH // NUM_WORKERS
L = 16

_mesh = plsc.VectorSubcoreMesh(core_axis_name="c", subcore_axis_name="s")


@functools.partial(
    pl.kernel,
    mesh=_mesh,
    compiler_params=pltpu.CompilerParams(
        needs_layout_passes=False, use_tc_tiling_on_sc=True),
    out_type=(
        jax.ShapeDtypeStruct((BATCH * DIM,), jnp.float32),
        jax.ShapeDtypeStruct((BATCH,), jnp.int32),
    ),
    scratch_types=[
        pltpu.VMEM((B_PER_W,), jnp.int32),
        pltpu.VMEM((B_PER_W, 2 * DIM), jnp.float32),
        pltpu.VMEM((B_PER_W * DIM,), jnp.float32),
        pltpu.VMEM((B_PER_W,), jnp.int32),
        pltpu.SemaphoreType.DMA,
        pltpu.SemaphoreType.DMA,
    ],
)
def _gather_kernel(nid_hbm, mem_hbm, last_hbm, mem_out_hbm, last_out_hbm,
                   idx_v, buf_v, out_v, last_v, sem_rows, sem_last):
    wid = lax.axis_index("s") * NUM_CORES + lax.axis_index("c")
    base = wid * B_PER_W
    pltpu.sync_copy(nid_hbm.at[pl.ds(base, B_PER_W)], idx_v)

    cp_rows = pltpu.async_copy(mem_hbm.at[idx_v], buf_v, sem_rows)
    cp_last = pltpu.async_copy(last_hbm.at[idx_v], last_v, sem_last)
    cp_rows.wait()

    iota = lax.iota(jnp.int32, L)

    def _extract(r):
        r_vec = jnp.full((L,), r, jnp.int32)
        for c in range(DIM // L):
            v = plsc.load_gather(buf_v, [r_vec, iota + c * L])
            out_v[pl.ds(r * DIM + c * L, L)] = v
    pl.loop(0, B_PER_W)(_extract)

    pltpu.sync_copy(out_v, mem_out_hbm.at[pl.ds(base * DIM, B_PER_W * DIM)])
    cp_last.wait()
    pltpu.sync_copy(last_v, last_out_hbm.at[pl.ds(base, B_PER_W)])


def kernel(n_id, memory, last_update):
    mem_pad = jnp.pad(memory, ((0, 0), (0, DIM)))
    mem_flat, last_out = _gather_kernel(
        n_id.astype(jnp.int32), mem_pad, last_update)
    return (mem_flat.reshape(BATCH, DIM), last_out,
            jnp.array(0, dtype=jnp.int32))


# speedup vs baseline: 1.5368x; 1.3707x over previous
import functools

import jax
import jax.numpy as jnp
from jax import lax
from jax.experimental import pallas as pl
from jax.experimental.pallas import tpu as pltpu
from jax.experimental.pallas import tpu_sc as plsc

BATCH = 16384
DIM = 64
NUM_CORES = 2
NUM_SUBCORES = 16
NUM_WORKERS = NUM_CORES * NUM_SUBCORES
B_PER_W = BATCH // NUM_WORKERS
L = 16
SLOTS = 8
GROUPS = B_PER_W // SLOTS

_mesh = plsc.VectorSubcoreMesh(core_axis_name="c", subcore_axis_name="s")


@functools.partial(
    pl.kernel,
    mesh=_mesh,
    compiler_params=pltpu.CompilerParams(
        needs_layout_passes=False, use_tc_tiling_on_sc=True),
    out_type=(
        jax.ShapeDtypeStruct((BATCH * DIM,), jnp.float32),
        jax.ShapeDtypeStruct((BATCH,), jnp.int32),
    ),
    scratch_types=[
        pltpu.VMEM((B_PER_W,), jnp.int32),
        pltpu.VMEM((2 * SLOTS, 8, DIM), jnp.float32),
        pltpu.VMEM((B_PER_W * DIM,), jnp.float32),
        pltpu.VMEM((B_PER_W,), jnp.int32),
        [pltpu.SemaphoreType.DMA] * (2 * SLOTS),
        pltpu.SemaphoreType.DMA,
    ],
)
def _gather_kernel(nid_hbm, mem_hbm, last_hbm, mem_out_hbm, last_out_hbm,
                   idx_v, buf_v, out_v, last_v, sems, sem_last):
    wid = lax.axis_index("s") * NUM_CORES + lax.axis_index("c")
    base = wid * B_PER_W
    pltpu.sync_copy(nid_hbm.at[pl.ds(base, B_PER_W)], idx_v)
    cp_last = pltpu.async_copy(last_hbm.at[idx_v], last_v, sem_last)

    mem3 = mem_hbm.reshape(125_000, 8, DIM)
    iota = lax.iota(jnp.int32, L)

    def _scalar_idx(j):
        blk = (j >> 4) << 4
        v = idx_v[pl.ds(blk, L)]
        return jnp.sum(jnp.where(iota == (j & 15), v, 0))

    def _fire(g, bank):
        for k in range(SLOTS):
            j = g * SLOTS + k
            oct = _scalar_idx(j) >> 3
            s = bank * SLOTS + k
            pltpu.async_copy(mem3.at[oct], buf_v.at[s], sems[s])

    def _drain(g, bank):
        for k in range(SLOTS):
            j = g * SLOTS + k
            s = bank * SLOTS + k
            pltpu.make_async_copy(
                mem3.at[0], buf_v.at[s], sems[s]).wait()
            jvec = jnp.full((L,), j, jnp.int32)
            nvec = plsc.load_gather(idx_v, [jvec])
            rowv = nvec & 7
            svec = jnp.full((L,), s, jnp.int32)
            for c in range(DIM // L):
                v = plsc.load_gather(buf_v, [svec, rowv, iota + c * L])
                out_v[pl.ds(j * DIM + c * L, L)] = v

    _fire(0, 0)

    def _body(t):
        g = 2 * t
        _fire(g + 1, 1)
        _drain(g, 0)
        _fire(g + 2, 0)
        _drain(g + 1, 1)
    pl.loop(0, GROUPS // 2 - 1)(_body)
    _fire(GROUPS - 1, 1)
    _drain(GROUPS - 2, 0)
    _drain(GROUPS - 1, 1)

    pltpu.sync_copy(out_v, mem_out_hbm.at[pl.ds(base * DIM, B_PER_W * DIM)])
    cp_last.wait()
    pltpu.sync_copy(last_v, last_out_hbm.at[pl.ds(base, B_PER_W)])


def kernel(n_id, memory, last_update):
    mem_flat, last_out = _gather_kernel(
        n_id.astype(jnp.int32), memory, last_update)
    return (mem_flat.reshape(BATCH, DIM), last_out,
            jnp.array(0, dtype=jnp.int32))
